# Initial kernel scaffold; baseline (speedup 1.0000x reference)
import jax
import jax.numpy as jnp
from jax.experimental import pallas as pl


def kernel(x, edge_index, edge_attr, c1_Wrel, c1_brel, c1_Wroot, c2_Wrel, c2_brel, c2_Wroot, c3_Wrel, c3_brel, c3_Wroot, c4_Wrel, c4_brel, c4_Wroot, f1_W, f1_b, r1_W, r1_b, f2_W, f2_b, r2_W, r2_b, f3_W, f3_b, r3_W, r3_b, f4_W, f4_b, r4_W, r4_b):
    raise NotImplementedError("write your pallas kernel here")


# speedup vs baseline: 1.5299x; 1.5299x over previous
import functools

import jax
import jax.numpy as jnp
from jax import lax
from jax.experimental import pallas as pl
from jax.experimental.pallas import tpu as pltpu
from jax.experimental.pallas import tpu_sc as plsc

F32 = jnp.float32
NC, NS = 2, 16
NW = NC * NS
EC = 80


def _mesh():
    return plsc.VectorSubcoreMesh(core_axis_name="c", subcore_axis_name="s")


@functools.cache
def _sc_gather2(n_nodes, d, n_edges):
    per_w = n_edges // NW
    iters = per_w // EC

    @functools.partial(
        pl.kernel,
        out_type=[jax.ShapeDtypeStruct((n_edges, d), F32),
                  jax.ShapeDtypeStruct((n_edges, d), F32)],
        mesh=_mesh(),
        scratch_types=[
            pltpu.VMEM((EC,), jnp.int32),
            pltpu.VMEM((EC,), jnp.int32),
            pltpu.VMEM((EC, d), F32),
            pltpu.VMEM((EC, d), F32),
            pltpu.SemaphoreType.DMA,
        ],
    )
    def k(table, src, dst, xi, xj, idx_s, idx_d, rows_i, rows_j, sem):
        wid = lax.axis_index("s") * NC + lax.axis_index("c")
        base = wid * per_w

        def body(i, carry):
            off = base + i * EC
            pltpu.sync_copy(src.at[pl.ds(off, EC)], idx_s)
            pltpu.sync_copy(dst.at[pl.ds(off, EC)], idx_d)
            a = pltpu.async_copy(table.at[idx_s], rows_i, sem)
            b = pltpu.async_copy(table.at[idx_d], rows_j, sem)
            a.wait()
            b.wait()
            pltpu.sync_copy(rows_i, xi.at[pl.ds(off, EC)])
            pltpu.sync_copy(rows_j, xj.at[pl.ds(off, EC)])
            return carry

        lax.fori_loop(0, iters, body, 0)

    return k


@functools.cache
def _sc_gather1(n_nodes, d, n_edges):
    per_w = n_edges // NW
    iters = per_w // EC

    @functools.partial(
        pl.kernel,
        out_type=jax.ShapeDtypeStruct((n_edges, d), F32),
        mesh=_mesh(),
        scratch_types=[
            pltpu.VMEM((EC,), jnp.int32),
            pltpu.VMEM((EC, d), F32),
            pltpu.SemaphoreType.DMA,
        ],
    )
    def k(table, idx, out, idx_v, rows, sem):
        wid = lax.axis_index("s") * NC + lax.axis_index("c")
        base = wid * per_w

        def body(i, carry):
            off = base + i * EC
            pltpu.sync_copy(idx.at[pl.ds(off, EC)], idx_v)
            pltpu.async_copy(table.at[idx_v], rows, sem).wait()
            pltpu.sync_copy(rows, out.at[pl.ds(off, EC)])
            return carry

        lax.fori_loop(0, iters, body, 0)

    return k


@functools.cache
def _sc_scatter(n_nodes, dm, n_edges):
    per_w = n_edges // NW
    iters = per_w // EC
    nchunks = n_nodes // EC
    per_tile_chunks = (nchunks + NS - 1) // NS

    @functools.partial(
        pl.kernel,
        out_type=jax.ShapeDtypeStruct((2 * n_nodes, dm), F32),
        mesh=_mesh(),
        scratch_types=[
            pltpu.VMEM((EC,), jnp.int32),
            pltpu.VMEM((EC, dm), F32),
            pltpu.VMEM_SHARED((n_nodes, dm), F32),
            pltpu.SemaphoreType.DMA,
        ],
    )
    def k(msg, dst, zeros, out, idx, rows, acc, sem):
        cid = lax.axis_index("c")
        sid = lax.axis_index("s")
        wid = sid * NC + cid
        base = wid * per_w

        pltpu.sync_copy(zeros, rows)

        def zbody(m, carry):
            ch = m * NS + sid

            @pl.when(ch < nchunks)
            def _():
                pltpu.sync_copy(rows, acc.at[pl.ds(ch * EC, EC)])

            return carry

        lax.fori_loop(0, per_tile_chunks, zbody, 0)
        plsc.subcore_barrier()

        def body(i, carry):
            off = base + i * EC
            pltpu.sync_copy(dst.at[pl.ds(off, EC)], idx)
            pltpu.sync_copy(msg.at[pl.ds(off, EC)], rows)
            pltpu.sync_copy(rows, acc.at[idx], add=True)
            return carry

        lax.fori_loop(0, iters, body, 0)
        plsc.subcore_barrier()

        def dbody(m, carry):
            ch = m * NS + sid

            @pl.when(ch < nchunks)
            def _():
                pltpu.sync_copy(acc.at[pl.ds(ch * EC, EC)], rows)
                pltpu.sync_copy(rows, out.at[pl.ds(cid * n_nodes + ch * EC, EC)])

            return carry

        lax.fori_loop(0, per_tile_chunks, dbody, 0)

    return k


def _dot(a, b):
    return lax.dot_general(a, b, (((1,), (0,)), ((), ())),
                           preferred_element_type=F32)


def _edge_call(xi, xj, ea2, wrelp, w1p, w2, w3, bf, wr, br, d_true):
    n_edges, d = xi.shape
    dm = d if wrelp is None else wrelp.shape[1]
    blk = 512
    grid = n_edges // blk
    use_w = wrelp is not None

    def body(*refs):
        if use_w:
            (xi_r, xj_r, ea_r, wrelp_r, w1_r, w2_r, w3_r, bf_r, wr_r, br_r,
             ean_r, msg_r) = refs
        else:
            (xi_r, xj_r, ea_r, w1_r, w2_r, w3_r, bf_r, wr_r, br_r,
             ean_r, msg_r) = refs
        xi_v = xi_r[...]
        ea_v = ea_r[...]
        dq = xi_v - xj_r[...]
        h = _dot(dq * dq, w1_r[...])
        h = h + xi_v[:, d_true - 1:d_true] * w2_r[...] + ea_v * w3_r[...] \
            + bf_r[...]
        h = jnp.maximum(h, 0.0)
        ean_r[...] = _dot(h, wr_r[...]) + br_r[...]
        g_v = _dot(xi_v, wrelp_r[...]) if use_w else xi_v
        msg_r[...] = g_v * ea_v

    nh = w1p.shape[1]
    full = lambda shape: pl.BlockSpec(shape, lambda i: (0, 0))
    espec = lambda w: pl.BlockSpec((blk, w), lambda i: (i, 0))
    in_specs = [espec(d), espec(d), espec(1)]
    args = [xi, xj, ea2]
    if use_w:
        in_specs.append(full(wrelp.shape))
        args.append(wrelp)
    in_specs += [full(w1p.shape), full((1, nh)), full((1, nh)),
                 full((1, nh)), full(wr.shape), full((1, 1))]
    args += [w1p, w2, w3, bf, wr, br]

    return pl.pallas_call(
        body,
        grid=(grid,),
        in_specs=in_specs,
        out_specs=[espec(1), espec(dm)],
        out_shape=[jax.ShapeDtypeStruct((n_edges, 1), F32),
                   jax.ShapeDtypeStruct((n_edges, dm), F32)],
    )(*args)


def _node_call(parts, x, wrel, wroot, brel, dm_true, relu, commute):
    n_nodes, d = x.shape
    dm = parts.shape[1]
    dout = wroot.shape[1]
    blk = 1000
    grid = n_nodes // blk

    def body(*refs):
        if commute:
            p0_r, p1_r, x_r, wroot_r, b_r, o_r = refs
            agg = (p0_r[...] + p1_r[...])[:, :dm_true]
        else:
            p0_r, p1_r, x_r, wrel_r, wroot_r, b_r, o_r = refs
            agg = _dot((p0_r[...] + p1_r[...])[:, :dm_true], wrel_r[...])
        t = agg + _dot(x_r[...], wroot_r[...]) + b_r[...]
        if relu:
            t = jnp.maximum(t, 0.0)
        o_r[...] = t

    full = lambda shape: pl.BlockSpec(shape, lambda i: (0, 0))
    in_specs = [pl.BlockSpec((blk, dm), lambda i: (i, 0)),
                pl.BlockSpec((blk, dm), lambda i, _g=grid: (i + _g, 0)),
                pl.BlockSpec((blk, d), lambda i: (i, 0))]
    args = [parts, parts, x]
    if not commute:
        in_specs.append(full(wrel.shape))
        args.append(wrel)
    in_specs += [full(wroot.shape), full((1, dout))]
    args += [wroot, brel.reshape(1, dout)]

    return pl.pallas_call(
        body,
        grid=(grid,),
        in_specs=in_specs,
        out_specs=pl.BlockSpec((blk, dout), lambda i: (i, 0)),
        out_shape=jax.ShapeDtypeStruct((n_nodes, dout), F32),
    )(*args)


def _matmul_call(x, w):
    n, d = x.shape
    dout = w.shape[1]
    blk = 1000
    grid = n // blk

    def body(x_r, w_r, o_r):
        o_r[...] = _dot(x_r[...], w_r[...])

    return pl.pallas_call(
        body,
        grid=(grid,),
        in_specs=[pl.BlockSpec((blk, d), lambda i: (i, 0)),
                  pl.BlockSpec(w.shape, lambda i: (0, 0))],
        out_specs=pl.BlockSpec((blk, dout), lambda i: (i, 0)),
        out_shape=jax.ShapeDtypeStruct((n, dout), F32),
    )(x, w)


def _edge_weights(fw, fb, rw, rb, d_true, dpad, nh):
    w1p = jnp.concatenate(
        [fw[:d_true - 1], jnp.zeros((dpad - d_true + 1, nh), F32)], axis=0)
    w2 = fw[d_true - 1:d_true]
    w3 = fw[d_true:d_true + 1]
    return w1p, w2, w3, fb.reshape(1, nh), rw, rb.reshape(1, 1)


def kernel(x, edge_index, edge_attr, c1_Wrel, c1_brel, c1_Wroot, c2_Wrel, c2_brel, c2_Wroot, c3_Wrel, c3_brel, c3_Wroot, c4_Wrel, c4_brel, c4_Wroot, f1_W, f1_b, r1_W, r1_b, f2_W, f2_b, r2_W, r2_b, f3_W, f3_b, r3_W, r3_b, f4_W, f4_b, r4_W, r4_b):
    n_nodes, _ = x.shape
    n_edges = edge_index.shape[1]
    src = edge_index[0]
    dst = edge_index[1]

    layers = [
        (c1_Wrel, c1_brel, c1_Wroot, f1_W, f1_b, r1_W, r1_b, True, False),
        (c2_Wrel, c2_brel, c2_Wroot, f2_W, f2_b, r2_W, r2_b, True, False),
        (c3_Wrel, c3_brel, c3_Wroot, f3_W, f3_b, r3_W, r3_b, True, True),
        (c4_Wrel, c4_brel, c4_Wroot, f4_W, f4_b, r4_W, r4_b, False, False),
    ]

    xc = x
    ea2 = edge_attr.reshape(n_edges, 1)
    for (wrel, brel, wroot, fw, fb, rw, rb, relu, commute) in layers:
        d = xc.shape[1]
        nh = fw.shape[1]
        dm = wrel.shape[1] if commute else d
        dpad = max(d, 128)
        xc_t = xc if d == dpad else jnp.pad(xc, ((0, 0), (0, dpad - d)))

        xi, xj = _sc_gather2(n_nodes, dpad, n_edges)(xc_t, src, dst)
        if commute:
            dmp = max(dm, 128)
            wrelp = jnp.pad(wrel, ((0, dpad - d), (0, dmp - dm)))
        else:
            wrelp = None
            dmp = dpad
        w1p, w2, w3, bf, wr, br = _edge_weights(fw, fb, rw, rb, d, dpad, nh)
        ean, msg = _edge_call(xi, xj, ea2, wrelp, w1p, w2, w3, bf, wr, br, d)
        parts = _sc_scatter(n_nodes, dmp, n_edges)(
            msg, dst, jnp.zeros((EC, dmp), F32))
        xc = _node_call(parts, xc, wrel, wroot, brel, dm, relu, commute)
        ea2 = ean

    return xc, ea2.reshape(n_edges)


# speedup vs baseline: 1.9386x; 1.2672x over previous
import functools

import jax
import jax.numpy as jnp
from jax import lax
from jax.experimental import pallas as pl
from jax.experimental.pallas import tpu as pltpu
from jax.experimental.pallas import tpu_sc as plsc

F32 = jnp.float32
NC, NS = 2, 16
NW = NC * NS
EC = 80


def _mesh():
    return plsc.VectorSubcoreMesh(core_axis_name="c", subcore_axis_name="s")


@functools.cache
def _sc_gather2(n_nodes, d, n_edges):
    per_w = n_edges // NW
    iters = per_w // EC

    @functools.partial(
        pl.kernel,
        out_type=[jax.ShapeDtypeStruct((n_edges, d), F32),
                  jax.ShapeDtypeStruct((n_edges, d), F32)],
        mesh=_mesh(),
        scratch_types=[
            pltpu.VMEM((iters, EC), jnp.int32),
            pltpu.VMEM((iters, EC), jnp.int32),
            pltpu.VMEM((2, EC, d), F32),
            pltpu.VMEM((2, EC, d), F32),
            pltpu.SemaphoreType.DMA((2,)),
            pltpu.SemaphoreType.DMA((2,)),
        ],
    )
    def k(table, src3, dst3, xi, xj, idxs, idxd, rows_i, rows_j, gsem, wsem):
        wid = lax.axis_index("s") * NC + lax.axis_index("c")
        base = wid * per_w
        pltpu.sync_copy(src3.at[wid], idxs)
        pltpu.sync_copy(dst3.at[wid], idxd)

        def gath(ch, slot):
            a = pltpu.make_async_copy(table.at[idxs.at[ch]], rows_i.at[slot],
                                      gsem.at[slot])
            b = pltpu.make_async_copy(table.at[idxd.at[ch]], rows_j.at[slot],
                                      gsem.at[slot])
            return a, b

        def wrt(ch, slot):
            a = pltpu.make_async_copy(rows_i.at[slot],
                                      xi.at[pl.ds(base + ch * EC, EC)],
                                      wsem.at[slot])
            b = pltpu.make_async_copy(rows_j.at[slot],
                                      xj.at[pl.ds(base + ch * EC, EC)],
                                      wsem.at[slot])
            return a, b

        a0, b0 = gath(0, 0)
        a0.start()
        b0.start()

        def body(kk, carry):
            s = lax.rem(kk, 2)
            ns = 1 - s

            @pl.when(kk >= 1)
            def _drain_writes():
                a, b = wrt(kk - 1, ns)
                a.wait()
                b.wait()

            @pl.when(kk + 1 < iters)
            def _next_gather():
                a, b = gath(kk + 1, ns)
                a.start()
                b.start()

            a, b = gath(kk, s)
            a.wait()
            b.wait()
            wa, wb = wrt(kk, s)
            wa.start()
            wb.start()
            return carry

        lax.fori_loop(0, iters, body, 0)
        a, b = wrt(iters - 1, (iters - 1) % 2)
        a.wait()
        b.wait()

    return k


@functools.cache
def _sc_scatter(n_nodes, dm, n_edges):
    per_w = n_edges // NW
    iters = per_w // EC
    nchunks = n_nodes // EC
    per_tile_chunks = (nchunks + NS - 1) // NS

    @functools.partial(
        pl.kernel,
        out_type=jax.ShapeDtypeStruct((2 * n_nodes, dm), F32),
        mesh=_mesh(),
        scratch_types=[
            pltpu.VMEM((iters, EC), jnp.int32),
            pltpu.VMEM((2, EC, dm), F32),
            pltpu.VMEM_SHARED((n_nodes, dm), F32),
            pltpu.SemaphoreType.DMA((2,)),
        ],
    )
    def k(msg, dst3, zeros, out, idx, rows, acc, lsem):
        cid = lax.axis_index("c")
        sid = lax.axis_index("s")
        wid = sid * NC + cid
        base = wid * per_w

        pltpu.sync_copy(zeros, rows.at[0])

        def zbody(m, carry):
            ch = m * NS + sid

            @pl.when(ch < nchunks)
            def _():
                pltpu.sync_copy(rows.at[0], acc.at[pl.ds(ch * EC, EC)])

            return carry

        lax.fori_loop(0, per_tile_chunks, zbody, 0)
        pltpu.sync_copy(dst3.at[wid], idx)
        plsc.subcore_barrier()

        def load(ch, slot):
            return pltpu.make_async_copy(msg.at[pl.ds(base + ch * EC, EC)],
                                         rows.at[slot], lsem.at[slot])

        load(0, 0).start()

        def body(kk, carry):
            s = lax.rem(kk, 2)
            ns = 1 - s
            load(kk, s).wait()

            @pl.when(kk + 1 < iters)
            def _next_load():
                load(kk + 1, ns).start()

            pltpu.sync_copy(rows.at[s], acc.at[idx.at[kk]], add=True)
            return carry

        lax.fori_loop(0, iters, body, 0)
        plsc.subcore_barrier()

        def dbody(m, carry):
            ch = m * NS + sid

            @pl.when(ch < nchunks)
            def _():
                pltpu.sync_copy(acc.at[pl.ds(ch * EC, EC)], rows.at[0])
                pltpu.sync_copy(rows.at[0],
                                out.at[pl.ds(cid * n_nodes + ch * EC, EC)])

            return carry

        lax.fori_loop(0, per_tile_chunks, dbody, 0)

    return k


def _dot(a, b):
    return lax.dot_general(a, b, (((1,), (0,)), ((), ())),
                           preferred_element_type=F32)


def _edge_call(xi, xj, ea2, wrelp, w1p, w2, w3, bf, wr, br, d_true):
    n_edges, d = xi.shape
    dm = d if wrelp is None else wrelp.shape[1]
    blk = 512
    grid = n_edges // blk
    use_w = wrelp is not None

    def body(*refs):
        if use_w:
            (xi_r, xj_r, ea_r, wrelp_r, w1_r, w2_r, w3_r, bf_r, wr_r, br_r,
             ean_r, msg_r) = refs
        else:
            (xi_r, xj_r, ea_r, w1_r, w2_r, w3_r, bf_r, wr_r, br_r,
             ean_r, msg_r) = refs
        xi_v = xi_r[...]
        ea_v = ea_r[...]
        dq = xi_v - xj_r[...]
        h = _dot(dq * dq, w1_r[...])
        h = h + xi_v[:, d_true - 1:d_true] * w2_r[...] + ea_v * w3_r[...] \
            + bf_r[...]
        h = jnp.maximum(h, 0.0)
        ean_r[...] = _dot(h, wr_r[...]) + br_r[...]
        g_v = _dot(xi_v, wrelp_r[...]) if use_w else xi_v
        msg_r[...] = g_v * ea_v

    nh = w1p.shape[1]
    full = lambda shape: pl.BlockSpec(shape, lambda i: (0, 0))
    espec = lambda w: pl.BlockSpec((blk, w), lambda i: (i, 0))
    in_specs = [espec(d), espec(d), espec(1)]
    args = [xi, xj, ea2]
    if use_w:
        in_specs.append(full(wrelp.shape))
        args.append(wrelp)
    in_specs += [full(w1p.shape), full((1, nh)), full((1, nh)),
                 full((1, nh)), full(wr.shape), full((1, 1))]
    args += [w1p, w2, w3, bf, wr, br]

    return pl.pallas_call(
        body,
        grid=(grid,),
        in_specs=in_specs,
        out_specs=[espec(1), espec(dm)],
        out_shape=[jax.ShapeDtypeStruct((n_edges, 1), F32),
                   jax.ShapeDtypeStruct((n_edges, dm), F32)],
    )(*args)


def _node_call(parts, x, wrel, wroot, brel, dm_true, relu, commute):
    n_nodes, d = x.shape
    dm = parts.shape[1]
    dout = wroot.shape[1]
    blk = 1000
    grid = n_nodes // blk

    def body(*refs):
        if commute:
            p0_r, p1_r, x_r, wroot_r, b_r, o_r = refs
            agg = (p0_r[...] + p1_r[...])[:, :dm_true]
        else:
            p0_r, p1_r, x_r, wrel_r, wroot_r, b_r, o_r = refs
            agg = _dot((p0_r[...] + p1_r[...])[:, :dm_true], wrel_r[...])
        t = agg + _dot(x_r[...], wroot_r[...]) + b_r[...]
        if relu:
            t = jnp.maximum(t, 0.0)
        o_r[...] = t

    full = lambda shape: pl.BlockSpec(shape, lambda i: (0, 0))
    in_specs = [pl.BlockSpec((blk, dm), lambda i: (i, 0)),
                pl.BlockSpec((blk, dm), lambda i, _g=grid: (i + _g, 0)),
                pl.BlockSpec((blk, d), lambda i: (i, 0))]
    args = [parts, parts, x]
    if not commute:
        in_specs.append(full(wrel.shape))
        args.append(wrel)
    in_specs += [full(wroot.shape), full((1, dout))]
    args += [wroot, brel.reshape(1, dout)]

    return pl.pallas_call(
        body,
        grid=(grid,),
        in_specs=in_specs,
        out_specs=pl.BlockSpec((blk, dout), lambda i: (i, 0)),
        out_shape=jax.ShapeDtypeStruct((n_nodes, dout), F32),
    )(*args)


def _matmul_call(x, w):
    n, d = x.shape
    dout = w.shape[1]
    blk = 1000
    grid = n // blk

    def body(x_r, w_r, o_r):
        o_r[...] = _dot(x_r[...], w_r[...])

    return pl.pallas_call(
        body,
        grid=(grid,),
        in_specs=[pl.BlockSpec((blk, d), lambda i: (i, 0)),
                  pl.BlockSpec(w.shape, lambda i: (0, 0))],
        out_specs=pl.BlockSpec((blk, dout), lambda i: (i, 0)),
        out_shape=jax.ShapeDtypeStruct((n, dout), F32),
    )(x, w)


def _edge_weights(fw, fb, rw, rb, d_true, dpad, nh):
    w1p = jnp.concatenate(
        [fw[:d_true - 1], jnp.zeros((dpad - d_true + 1, nh), F32)], axis=0)
    w2 = fw[d_true - 1:d_true]
    w3 = fw[d_true:d_true + 1]
    return w1p, w2, w3, fb.reshape(1, nh), rw, rb.reshape(1, 1)


def kernel(x, edge_index, edge_attr, c1_Wrel, c1_brel, c1_Wroot, c2_Wrel, c2_brel, c2_Wroot, c3_Wrel, c3_brel, c3_Wroot, c4_Wrel, c4_brel, c4_Wroot, f1_W, f1_b, r1_W, r1_b, f2_W, f2_b, r2_W, r2_b, f3_W, f3_b, r3_W, r3_b, f4_W, f4_b, r4_W, r4_b):
    n_nodes, _ = x.shape
    n_edges = edge_index.shape[1]
    src3 = edge_index[0].reshape(NW, -1, EC)
    dst3 = edge_index[1].reshape(NW, -1, EC)

    layers = [
        (c1_Wrel, c1_brel, c1_Wroot, f1_W, f1_b, r1_W, r1_b, True, False),
        (c2_Wrel, c2_brel, c2_Wroot, f2_W, f2_b, r2_W, r2_b, True, False),
        (c3_Wrel, c3_brel, c3_Wroot, f3_W, f3_b, r3_W, r3_b, True, True),
        (c4_Wrel, c4_brel, c4_Wroot, f4_W, f4_b, r4_W, r4_b, False, False),
    ]

    xc = x
    ea2 = edge_attr.reshape(n_edges, 1)
    for (wrel, brel, wroot, fw, fb, rw, rb, relu, commute) in layers:
        d = xc.shape[1]
        nh = fw.shape[1]
        dm = wrel.shape[1] if commute else d
        dpad = max(d, 128)
        xc_t = xc if d == dpad else jnp.pad(xc, ((0, 0), (0, dpad - d)))

        xi, xj = _sc_gather2(n_nodes, dpad, n_edges)(xc_t, src3, dst3)
        if commute:
            dmp = max(dm, 128)
            wrelp = jnp.pad(wrel, ((0, dpad - d), (0, dmp - dm)))
        else:
            wrelp = None
            dmp = dpad
        w1p, w2, w3, bf, wr, br = _edge_weights(fw, fb, rw, rb, d, dpad, nh)
        ean, msg = _edge_call(xi, xj, ea2, wrelp, w1p, w2, w3, bf, wr, br, d)
        parts = _sc_scatter(n_nodes, dmp, n_edges)(
            msg, dst3, jnp.zeros((EC, dmp), F32))
        xc = _node_call(parts, xc, wrel, wroot, brel, dm, relu, commute)
        ea2 = ean

    return xc, ea2.reshape(n_edges)
